# baseline (speedup 1.0000x reference)
import functools

import jax
import jax.numpy as jnp
import numpy as np
from jax import lax
from jax.experimental import pallas as pl
from jax.experimental.pallas import tpu as pltpu
from jax.experimental.pallas import tpu_sc as plsc

_B = 8
_N = 500000
_H = 128
_W = 128
_NB = 16
_K = 5
_P = _H * _W
_NBP = _NB * _P

_NC = 2
_NS = 16
_L = 16
_NR = 4
_RNG = _NBP // _NR
_CH = 20864
_NCHUNK = 24


_NPAD = 500736
_IDB = _B * _NPAD // 24


def _idx_body(x_ref, y_ref, t_ref, out_ref):
    x = x_ref[...]
    y = y_ref[...]
    t = t_ref[...]
    xi = jnp.floor(jnp.clip(x, 0.0, 127.0))
    yi = jnp.floor(jnp.clip(y, 0.0, 127.0))
    tb = jnp.minimum(jnp.floor(jnp.clip(t, 0.0, 1.0) * 16.0), 15.0)
    idx = (tb * 16384.0 + xi * 128.0 + yi).astype(jnp.int32)
    pid = pl.program_id(0)
    wpos = (pid % 3) * _IDB + lax.broadcasted_iota(jnp.int32, (_IDB,), 0)
    out_ref[...] = jnp.where(wpos < _N, idx, -1)


def _compute_idx(events):
    ev_t = events.transpose(0, 2, 1)

    def plane(c):
        p2 = jnp.pad(ev_t[:, c, :], ((0, 0), (0, _NPAD - _N)))
        return p2.reshape(_B * _NPAD)

    bs = pl.BlockSpec((_IDB,), lambda i: (i,))
    return pl.pallas_call(
        _idx_body,
        grid=(_B * _NPAD // _IDB,),
        in_specs=[bs, bs, bs],
        out_specs=bs,
        out_shape=jax.ShapeDtypeStruct((_B * _NPAD,), jnp.int32),
    )(plane(0), plane(1), plane(2))


def _hist_sc_body(idx_hbm, out_hbm, buf0, buf1, hist, sem0, sem1):
    cid = lax.axis_index("c")
    sid = lax.axis_index("s")
    wid = sid * _NC + cid
    b = wid // _NR
    r = wid % _NR
    base = b * _NPAD
    rbase = r * _RNG

    zero16 = jnp.zeros((_L,), jnp.float32)

    @plsc.parallel_loop(0, _RNG // _L, unroll=8)
    def _zero(i):
        hist[pl.ds(i * _L, _L)] = zero16

    ones = jnp.ones((_L,), jnp.float32)

    def start(c, buf, sem):
        off = pl.multiple_of(base + c * _CH, 8)
        pltpu.async_copy(idx_hbm.at[pl.ds(off, _CH)], buf, sem)

    def wait(buf, sem):
        pltpu.make_async_copy(idx_hbm.at[pl.ds(0, _CH)], buf, sem).wait()

    def process(buf):
        @plsc.parallel_loop(0, _CH // _L, unroll=8)
        def _scat(i):
            v = buf[pl.ds(i * _L, _L)]
            local = v - rbase
            mask = lax.bitcast_convert_type(local, jnp.uint32) < jnp.uint32(_RNG)
            plsc.addupdate_scatter(hist, [local], ones, mask=mask)

    start(0, buf0, sem0)
    npair = _NCHUNK // 2

    def pair(p, carry):
        start(2 * p + 1, buf1, sem1)
        wait(buf0, sem0)
        process(buf0)

        @pl.when(p + 1 < npair)
        def _pref():
            start(2 * p + 2, buf0, sem0)

        wait(buf1, sem1)
        process(buf1)
        return carry

    lax.fori_loop(0, npair, pair, 0)

    pltpu.sync_copy(
        hist, out_hbm.at[pl.ds(pl.multiple_of(b * _NBP + rbase, 8), _RNG)])


@functools.lru_cache(maxsize=1)
def _hist_sc():
    mesh = plsc.VectorSubcoreMesh(
        core_axis_name="c", subcore_axis_name="s",
        num_cores=_NC, num_subcores=_NS)
    return pl.kernel(
        _hist_sc_body,
        out_type=jax.ShapeDtypeStruct((_B * _NBP,), jnp.float32),
        mesh=mesh,
        scratch_types=[
            pltpu.VMEM((_CH,), jnp.int32),
            pltpu.VMEM((_CH,), jnp.int32),
            pltpu.VMEM((_RNG,), jnp.float32),
            pltpu.SemaphoreType.DMA,
            pltpu.SemaphoreType.DMA,
        ],
        compiler_params=pltpu.CompilerParams(needs_layout_passes=False),
    )


_SIG = 5.0 / 6.0
_GAUSS = np.exp(-(np.arange(_K, dtype=np.float32) - 2.0) ** 2
                / np.float32(2.0 * _SIG * _SIG)).astype(np.float32)
_GAUSS = (_GAUSS / _GAUSS.sum()).astype(np.float32)
_G0 = float(_GAUSS[2])
_G1 = float(_GAUSS[1])
_G2 = float(_GAUSS[0])


def _post_body(h_ref, pnw_ref, pnb_ref, gnw_ref, gnb_ref, out_ref):
    x = h_ref[0]
    z1 = jnp.zeros((1, _P), jnp.float32)
    z2 = jnp.zeros((2, _P), jnp.float32)
    up1 = jnp.concatenate([x[1:], z1], axis=0)
    up2 = jnp.concatenate([x[2:], z2], axis=0)
    dn1 = jnp.concatenate([z1, x[:-1]], axis=0)
    dn2 = jnp.concatenate([z2, x[:-2]], axis=0)
    sm = _G0 * x + _G1 * (up1 + dn1) + _G2 * (up2 + dn2)
    mu = jnp.mean(sm, axis=1, keepdims=True)
    d = sm - mu
    var = jnp.mean(d * d, axis=1, keepdims=True)
    y = d * lax.rsqrt(var + 1e-5) * pnw_ref[...] + pnb_ref[...]
    mu2 = jnp.mean(y)
    d2 = y - mu2
    var2 = jnp.mean(d2 * d2)
    out_ref[0] = d2 * lax.rsqrt(var2 + 1e-5) * gnw_ref[...] + gnb_ref[...]


def _postprocess(hist, pn_w, pn_b, gn_w, gn_b):
    return pl.pallas_call(
        _post_body,
        grid=(_B,),
        in_specs=[
            pl.BlockSpec((1, _NB, _P), lambda b: (b, 0, 0)),
            pl.BlockSpec((1, _P), lambda b: (0, 0)),
            pl.BlockSpec((1, _P), lambda b: (0, 0)),
            pl.BlockSpec((_NB, _P), lambda b: (0, 0)),
            pl.BlockSpec((_NB, _P), lambda b: (0, 0)),
        ],
        out_specs=pl.BlockSpec((1, _NB, _P), lambda b: (b, 0, 0)),
        out_shape=jax.ShapeDtypeStruct((_B, _NB, _P), jnp.float32),
    )(hist.reshape(_B, _NB, _P), pn_w.reshape(1, _P), pn_b.reshape(1, _P),
      gn_w, gn_b)


def kernel(events, pn_w, pn_b, gn_w, gn_b):
    idx = _compute_idx(events)
    hist = _hist_sc()(idx)
    return _postprocess(hist, pn_w, pn_b, gn_w, gn_b)


# speedup vs baseline: 1.0002x; 1.0002x over previous
import functools

import jax
import jax.numpy as jnp
import numpy as np
from jax import lax
from jax.experimental import pallas as pl
from jax.experimental.pallas import tpu as pltpu
from jax.experimental.pallas import tpu_sc as plsc

_B = 8
_N = 500000
_H = 128
_W = 128
_NB = 16
_K = 5
_P = _H * _W
_NBP = _NB * _P

_NC = 2
_NS = 16
_L = 16
_NR = 4
_RNG = _NBP // _NR
_CROWS = 24
_NCH = 163


_KR = 3912
_NPAD = _KR * 128
_KB = 652


def _idx_body(x_ref, y_ref, t_ref, out_ref):
    x = x_ref[...]
    y = y_ref[...]
    t = t_ref[...]
    xi = jnp.floor(jnp.clip(x, 0.0, 127.0))
    yi = jnp.floor(jnp.clip(y, 0.0, 127.0))
    tb = jnp.minimum(jnp.floor(jnp.clip(t, 0.0, 1.0) * 16.0), 15.0)
    idx = (tb * 16384.0 + xi * 128.0 + yi).astype(jnp.int32)
    k = pl.program_id(0) * _KB + lax.broadcasted_iota(jnp.int32, x.shape, 0)
    lane = lax.broadcasted_iota(jnp.int32, x.shape, 2)
    valid = k * 128 + lane < _N
    out_ref[...] = jnp.where(valid, idx, -1)


def _compute_idx(events):
    ev_t = events.transpose(0, 2, 1)

    def plane(c):
        p2 = jnp.pad(ev_t[:, c, :], ((0, 0), (0, _NPAD - _N)))
        return p2.reshape(_B, _KR, 128).transpose(1, 0, 2)

    bs = pl.BlockSpec((_KB, 8, 128), lambda i: (i, 0, 0))
    out = pl.pallas_call(
        _idx_body,
        grid=(_KR // _KB,),
        in_specs=[bs, bs, bs],
        out_specs=bs,
        out_shape=jax.ShapeDtypeStruct((_KR, 8, 128), jnp.int32),
    )(plane(0), plane(1), plane(2))
    return out.reshape(_KR * _B, 128)


def _hist_sc_body(idx_hbm, out_hbm, rb0, rb1, buf0, buf1, hist, sem0, sem1):
    cid = lax.axis_index("c")
    sid = lax.axis_index("s")
    wid = sid * _NC + cid
    b = wid // _NR
    r = wid % _NR
    rbase = r * _RNG

    zero16 = jnp.zeros((_L,), jnp.float32)

    @plsc.parallel_loop(0, _RNG // _L, unroll=8)
    def _zero(i):
        hist[pl.ds(i * _L, _L)] = zero16

    ones = jnp.ones((_L,), jnp.float32)
    vec8 = lax.iota(jnp.int32, _L) * 8

    def start(c, rbuf, buf, sem):
        base_row = 192 * c + b
        rbuf[pl.ds(0, _L)] = vec8 + base_row
        rbuf[pl.ds(8, _L)] = vec8 + (base_row + 64)
        pltpu.async_copy(idx_hbm.at[rbuf], buf, sem)

    def wait(rbuf, buf, sem):
        pltpu.make_async_copy(idx_hbm.at[rbuf], buf, sem).wait()

    def process(buf):
        @plsc.parallel_loop(0, _CROWS * 8, unroll=8)
        def _scat(i):
            v = buf[i >> 3, pl.ds((i & 7) * _L, _L)]
            local = v - rbase
            mask = lax.bitcast_convert_type(local, jnp.uint32) < jnp.uint32(_RNG)
            plsc.addupdate_scatter(hist, [local], ones, mask=mask)

    start(0, rb0, buf0, sem0)

    def pair(p, carry):
        start(2 * p + 1, rb1, buf1, sem1)
        wait(rb0, buf0, sem0)
        process(buf0)
        start(2 * p + 2, rb0, buf0, sem0)
        wait(rb1, buf1, sem1)
        process(buf1)
        return carry

    lax.fori_loop(0, (_NCH - 1) // 2, pair, 0)
    wait(rb0, buf0, sem0)
    process(buf0)

    pltpu.sync_copy(
        hist, out_hbm.at[pl.ds(pl.multiple_of(b * _NBP + rbase, 8), _RNG)])


@functools.lru_cache(maxsize=1)
def _hist_sc():
    mesh = plsc.VectorSubcoreMesh(
        core_axis_name="c", subcore_axis_name="s",
        num_cores=_NC, num_subcores=_NS)
    return pl.kernel(
        _hist_sc_body,
        out_type=jax.ShapeDtypeStruct((_B * _NBP,), jnp.float32),
        mesh=mesh,
        scratch_types=[
            pltpu.VMEM((_CROWS,), jnp.int32),
            pltpu.VMEM((_CROWS,), jnp.int32),
            pltpu.VMEM((_CROWS, 128), jnp.int32),
            pltpu.VMEM((_CROWS, 128), jnp.int32),
            pltpu.VMEM((_RNG,), jnp.float32),
            pltpu.SemaphoreType.DMA,
            pltpu.SemaphoreType.DMA,
        ],
        compiler_params=pltpu.CompilerParams(needs_layout_passes=False),
    )


_SIG = 5.0 / 6.0
_GAUSS = np.exp(-(np.arange(_K, dtype=np.float32) - 2.0) ** 2
                / np.float32(2.0 * _SIG * _SIG)).astype(np.float32)
_GAUSS = (_GAUSS / _GAUSS.sum()).astype(np.float32)
_G0 = float(_GAUSS[2])
_G1 = float(_GAUSS[1])
_G2 = float(_GAUSS[0])


def _post_body(h_ref, pnw_ref, pnb_ref, gnw_ref, gnb_ref, out_ref):
    x = h_ref[0]
    z1 = jnp.zeros((1, _P), jnp.float32)
    z2 = jnp.zeros((2, _P), jnp.float32)
    up1 = jnp.concatenate([x[1:], z1], axis=0)
    up2 = jnp.concatenate([x[2:], z2], axis=0)
    dn1 = jnp.concatenate([z1, x[:-1]], axis=0)
    dn2 = jnp.concatenate([z2, x[:-2]], axis=0)
    sm = _G0 * x + _G1 * (up1 + dn1) + _G2 * (up2 + dn2)
    mu = jnp.mean(sm, axis=1, keepdims=True)
    d = sm - mu
    var = jnp.mean(d * d, axis=1, keepdims=True)
    y = d * lax.rsqrt(var + 1e-5) * pnw_ref[...] + pnb_ref[...]
    mu2 = jnp.mean(y)
    d2 = y - mu2
    var2 = jnp.mean(d2 * d2)
    out_ref[0] = d2 * lax.rsqrt(var2 + 1e-5) * gnw_ref[...] + gnb_ref[...]


def _postprocess(hist, pn_w, pn_b, gn_w, gn_b):
    return pl.pallas_call(
        _post_body,
        grid=(_B,),
        in_specs=[
            pl.BlockSpec((1, _NB, _P), lambda b: (b, 0, 0)),
            pl.BlockSpec((1, _P), lambda b: (0, 0)),
            pl.BlockSpec((1, _P), lambda b: (0, 0)),
            pl.BlockSpec((_NB, _P), lambda b: (0, 0)),
            pl.BlockSpec((_NB, _P), lambda b: (0, 0)),
        ],
        out_specs=pl.BlockSpec((1, _NB, _P), lambda b: (b, 0, 0)),
        out_shape=jax.ShapeDtypeStruct((_B, _NB, _P), jnp.float32),
    )(hist.reshape(_B, _NB, _P), pn_w.reshape(1, _P), pn_b.reshape(1, _P),
      gn_w, gn_b)


def kernel(events, pn_w, pn_b, gn_w, gn_b):
    idx = _compute_idx(events)
    hist = _hist_sc()(idx)
    return _postprocess(hist, pn_w, pn_b, gn_w, gn_b)


# speedup vs baseline: 1.1524x; 1.1522x over previous
import functools

import jax
import jax.numpy as jnp
import numpy as np
from jax import lax
from jax.experimental import pallas as pl
from jax.experimental.pallas import tpu as pltpu
from jax.experimental.pallas import tpu_sc as plsc

_B = 8
_N = 500000
_H = 128
_W = 128
_NB = 16
_K = 5
_P = _H * _W
_NBP = _NB * _P

_NC = 2
_NS = 16
_L = 16
_NR = 4
_RNG = _NBP // _NR
_CH = 20864
_NCHUNK = 24


_KR = 3912
_NPAD = _KR * 128
_KB = 1304


def _idx_body(x_ref, y_ref, t_ref, out_ref):
    x = x_ref[...]
    y = y_ref[...]
    t = t_ref[...]
    xi = jnp.floor(jnp.clip(x, 0.0, 127.0))
    yi = jnp.floor(jnp.clip(y, 0.0, 127.0))
    tb = jnp.minimum(jnp.floor(jnp.clip(t, 0.0, 1.0) * 16.0), 15.0)
    idx = (tb * 16384.0 + xi * 128.0 + yi).astype(jnp.int32)
    k = pl.program_id(0) * _KB + lax.broadcasted_iota(jnp.int32, x.shape, 0)
    lane = lax.broadcasted_iota(jnp.int32, x.shape, 2)
    valid = k * 128 + lane < _N
    out_ref[...] = jnp.transpose(jnp.where(valid, idx, -1), (1, 0, 2))


def _compute_idx(events):
    ev_t = events.transpose(0, 2, 1)

    def plane(c):
        p2 = jnp.pad(ev_t[:, c, :], ((0, 0), (0, _NPAD - _N)))
        return p2.reshape(_B, _KR, 128).transpose(1, 0, 2)

    bs = pl.BlockSpec((_KB, 8, 128), lambda j: (j, 0, 0))
    out = pl.pallas_call(
        _idx_body,
        grid=(_KR // _KB,),
        in_specs=[bs, bs, bs],
        out_specs=pl.BlockSpec((_B, _KB, 128), lambda j: (0, j, 0)),
        out_shape=jax.ShapeDtypeStruct((_B, _KR, 128), jnp.int32),
        compiler_params=pltpu.CompilerParams(vmem_limit_bytes=100 * 1024 * 1024),
    )(plane(0), plane(1), plane(2))
    return out.reshape(_B * _NPAD)


def _hist_sc_body(idx_hbm, out_hbm, buf0, buf1, hist, sem0, sem1):
    cid = lax.axis_index("c")
    sid = lax.axis_index("s")
    wid = sid * _NC + cid
    b = wid // _NR
    r = wid % _NR
    base = b * _NPAD
    rbase = r * _RNG

    zero16 = jnp.zeros((_L,), jnp.float32)

    @plsc.parallel_loop(0, _RNG // _L, unroll=8)
    def _zero(i):
        hist[pl.ds(i * _L, _L)] = zero16

    ones = jnp.ones((_L,), jnp.float32)

    def start(c, buf, sem):
        off = pl.multiple_of(base + c * _CH, 8)
        pltpu.async_copy(idx_hbm.at[pl.ds(off, _CH)], buf, sem)

    def wait(buf, sem):
        pltpu.make_async_copy(idx_hbm.at[pl.ds(0, _CH)], buf, sem).wait()

    def process(buf):
        @plsc.parallel_loop(0, _CH // _L, unroll=8)
        def _scat(i):
            v = buf[pl.ds(i * _L, _L)]
            local = v - rbase
            mask = lax.bitcast_convert_type(local, jnp.uint32) < jnp.uint32(_RNG)
            plsc.addupdate_scatter(hist, [local], ones, mask=mask)

    start(0, buf0, sem0)
    npair = _NCHUNK // 2

    def pair(p, carry):
        start(2 * p + 1, buf1, sem1)
        wait(buf0, sem0)
        process(buf0)

        @pl.when(p + 1 < npair)
        def _pref():
            start(2 * p + 2, buf0, sem0)

        wait(buf1, sem1)
        process(buf1)
        return carry

    lax.fori_loop(0, npair, pair, 0)

    pltpu.sync_copy(
        hist, out_hbm.at[pl.ds(pl.multiple_of(b * _NBP + rbase, 8), _RNG)])


@functools.lru_cache(maxsize=1)
def _hist_sc():
    mesh = plsc.VectorSubcoreMesh(
        core_axis_name="c", subcore_axis_name="s",
        num_cores=_NC, num_subcores=_NS)
    return pl.kernel(
        _hist_sc_body,
        out_type=jax.ShapeDtypeStruct((_B * _NBP,), jnp.float32),
        mesh=mesh,
        scratch_types=[
            pltpu.VMEM((_CH,), jnp.int32),
            pltpu.VMEM((_CH,), jnp.int32),
            pltpu.VMEM((_RNG,), jnp.float32),
            pltpu.SemaphoreType.DMA,
            pltpu.SemaphoreType.DMA,
        ],
        compiler_params=pltpu.CompilerParams(needs_layout_passes=False),
    )


_SIG = 5.0 / 6.0
_GAUSS = np.exp(-(np.arange(_K, dtype=np.float32) - 2.0) ** 2
                / np.float32(2.0 * _SIG * _SIG)).astype(np.float32)
_GAUSS = (_GAUSS / _GAUSS.sum()).astype(np.float32)
_G0 = float(_GAUSS[2])
_G1 = float(_GAUSS[1])
_G2 = float(_GAUSS[0])


def _post_body(h_ref, pnw_ref, pnb_ref, gnw_ref, gnb_ref, out_ref):
    x = h_ref[0]
    z1 = jnp.zeros((1, _P), jnp.float32)
    z2 = jnp.zeros((2, _P), jnp.float32)
    up1 = jnp.concatenate([x[1:], z1], axis=0)
    up2 = jnp.concatenate([x[2:], z2], axis=0)
    dn1 = jnp.concatenate([z1, x[:-1]], axis=0)
    dn2 = jnp.concatenate([z2, x[:-2]], axis=0)
    sm = _G0 * x + _G1 * (up1 + dn1) + _G2 * (up2 + dn2)
    mu = jnp.mean(sm, axis=1, keepdims=True)
    d = sm - mu
    var = jnp.mean(d * d, axis=1, keepdims=True)
    y = d * lax.rsqrt(var + 1e-5) * pnw_ref[...] + pnb_ref[...]
    mu2 = jnp.mean(y)
    d2 = y - mu2
    var2 = jnp.mean(d2 * d2)
    out_ref[0] = d2 * lax.rsqrt(var2 + 1e-5) * gnw_ref[...] + gnb_ref[...]


def _postprocess(hist, pn_w, pn_b, gn_w, gn_b):
    return pl.pallas_call(
        _post_body,
        grid=(_B,),
        in_specs=[
            pl.BlockSpec((1, _NB, _P), lambda b: (b, 0, 0)),
            pl.BlockSpec((1, _P), lambda b: (0, 0)),
            pl.BlockSpec((1, _P), lambda b: (0, 0)),
            pl.BlockSpec((_NB, _P), lambda b: (0, 0)),
            pl.BlockSpec((_NB, _P), lambda b: (0, 0)),
        ],
        out_specs=pl.BlockSpec((1, _NB, _P), lambda b: (b, 0, 0)),
        out_shape=jax.ShapeDtypeStruct((_B, _NB, _P), jnp.float32),
    )(hist.reshape(_B, _NB, _P), pn_w.reshape(1, _P), pn_b.reshape(1, _P),
      gn_w, gn_b)


def kernel(events, pn_w, pn_b, gn_w, gn_b):
    idx = _compute_idx(events)
    hist = _hist_sc()(idx)
    return _postprocess(hist, pn_w, pn_b, gn_w, gn_b)


# speedup vs baseline: 1.4500x; 1.2582x over previous
import functools

import jax
import jax.numpy as jnp
import numpy as np
from jax import lax
from jax.experimental import pallas as pl
from jax.experimental.pallas import tpu as pltpu
from jax.experimental.pallas import tpu_sc as plsc

_B = 8
_N = 500000
_H = 128
_W = 128
_NB = 16
_K = 5
_P = _H * _W
_NBP = _NB * _P

_NC = 2
_NS = 16
_L = 16
_NR = 4
_RNG = _NBP // _NR
_CH = 20864
_NCHUNK = 24


_KR = 3912
_NPAD = _KR * 128
_KB = 1304


def _idx_body(ev_ref, out_ref):
    ev = ev_ref[0]
    x = ev[:, 0, :]
    y = ev[:, 1, :]
    t = ev[:, 2, :]
    xi = jnp.floor(jnp.clip(x, 0.0, 127.0))
    yi = jnp.floor(jnp.clip(y, 0.0, 127.0))
    tb = jnp.minimum(jnp.floor(jnp.clip(t, 0.0, 1.0) * 16.0), 15.0)
    idx = (tb * 16384.0 + xi * 128.0 + yi).astype(jnp.int32)
    k = pl.program_id(1) * _KB + lax.broadcasted_iota(jnp.int32, x.shape, 0)
    lane = lax.broadcasted_iota(jnp.int32, x.shape, 1)
    valid = k * 128 + lane < _N
    out_ref[0] = jnp.where(valid, idx, -1)


def _compute_idx(events):
    ev_p = jnp.pad(events, ((0, 0), (0, _NPAD - _N), (0, 0)))
    ev4 = ev_p.transpose(0, 2, 1).reshape(_B, 4, _KR, 128).transpose(0, 2, 1, 3)
    out = pl.pallas_call(
        _idx_body,
        grid=(_B, _KR // _KB),
        in_specs=[pl.BlockSpec((1, _KB, 4, 128), lambda b, j: (b, j, 0, 0))],
        out_specs=pl.BlockSpec((1, _KB, 128), lambda b, j: (b, j, 0)),
        out_shape=jax.ShapeDtypeStruct((_B, _KR, 128), jnp.int32),
    )(ev4)
    return out.reshape(_B * _NPAD)


def _hist_sc_body(idx_hbm, out_hbm, buf0, buf1, hist, sem0, sem1):
    cid = lax.axis_index("c")
    sid = lax.axis_index("s")
    wid = sid * _NC + cid
    b = wid // _NR
    r = wid % _NR
    base = b * _NPAD
    rbase = r * _RNG

    zero16 = jnp.zeros((_L,), jnp.float32)

    @plsc.parallel_loop(0, _RNG // _L, unroll=8)
    def _zero(i):
        hist[pl.ds(i * _L, _L)] = zero16

    ones = jnp.ones((_L,), jnp.float32)

    def start(c, buf, sem):
        off = pl.multiple_of(base + c * _CH, 8)
        pltpu.async_copy(idx_hbm.at[pl.ds(off, _CH)], buf, sem)

    def wait(buf, sem):
        pltpu.make_async_copy(idx_hbm.at[pl.ds(0, _CH)], buf, sem).wait()

    def process(buf):
        @plsc.parallel_loop(0, _CH // _L, unroll=8)
        def _scat(i):
            v = buf[pl.ds(i * _L, _L)]
            local = v - rbase
            mask = lax.bitcast_convert_type(local, jnp.uint32) < jnp.uint32(_RNG)
            plsc.addupdate_scatter(hist, [local], ones, mask=mask)

    start(0, buf0, sem0)
    npair = _NCHUNK // 2

    def pair(p, carry):
        start(2 * p + 1, buf1, sem1)
        wait(buf0, sem0)
        process(buf0)

        @pl.when(p + 1 < npair)
        def _pref():
            start(2 * p + 2, buf0, sem0)

        wait(buf1, sem1)
        process(buf1)
        return carry

    lax.fori_loop(0, npair, pair, 0)

    pltpu.sync_copy(
        hist, out_hbm.at[pl.ds(pl.multiple_of(b * _NBP + rbase, 8), _RNG)])


@functools.lru_cache(maxsize=1)
def _hist_sc():
    mesh = plsc.VectorSubcoreMesh(
        core_axis_name="c", subcore_axis_name="s",
        num_cores=_NC, num_subcores=_NS)
    return pl.kernel(
        _hist_sc_body,
        out_type=jax.ShapeDtypeStruct((_B * _NBP,), jnp.float32),
        mesh=mesh,
        scratch_types=[
            pltpu.VMEM((_CH,), jnp.int32),
            pltpu.VMEM((_CH,), jnp.int32),
            pltpu.VMEM((_RNG,), jnp.float32),
            pltpu.SemaphoreType.DMA,
            pltpu.SemaphoreType.DMA,
        ],
        compiler_params=pltpu.CompilerParams(needs_layout_passes=False),
    )


_SIG = 5.0 / 6.0
_GAUSS = np.exp(-(np.arange(_K, dtype=np.float32) - 2.0) ** 2
                / np.float32(2.0 * _SIG * _SIG)).astype(np.float32)
_GAUSS = (_GAUSS / _GAUSS.sum()).astype(np.float32)
_G0 = float(_GAUSS[2])
_G1 = float(_GAUSS[1])
_G2 = float(_GAUSS[0])


def _post_body(h_ref, pnw_ref, pnb_ref, gnw_ref, gnb_ref, out_ref):
    x = h_ref[0]
    z1 = jnp.zeros((1, _P), jnp.float32)
    z2 = jnp.zeros((2, _P), jnp.float32)
    up1 = jnp.concatenate([x[1:], z1], axis=0)
    up2 = jnp.concatenate([x[2:], z2], axis=0)
    dn1 = jnp.concatenate([z1, x[:-1]], axis=0)
    dn2 = jnp.concatenate([z2, x[:-2]], axis=0)
    sm = _G0 * x + _G1 * (up1 + dn1) + _G2 * (up2 + dn2)
    mu = jnp.mean(sm, axis=1, keepdims=True)
    d = sm - mu
    var = jnp.mean(d * d, axis=1, keepdims=True)
    y = d * lax.rsqrt(var + 1e-5) * pnw_ref[...] + pnb_ref[...]
    mu2 = jnp.mean(y)
    d2 = y - mu2
    var2 = jnp.mean(d2 * d2)
    out_ref[0] = d2 * lax.rsqrt(var2 + 1e-5) * gnw_ref[...] + gnb_ref[...]


def _postprocess(hist, pn_w, pn_b, gn_w, gn_b):
    return pl.pallas_call(
        _post_body,
        grid=(_B,),
        in_specs=[
            pl.BlockSpec((1, _NB, _P), lambda b: (b, 0, 0)),
            pl.BlockSpec((1, _P), lambda b: (0, 0)),
            pl.BlockSpec((1, _P), lambda b: (0, 0)),
            pl.BlockSpec((_NB, _P), lambda b: (0, 0)),
            pl.BlockSpec((_NB, _P), lambda b: (0, 0)),
        ],
        out_specs=pl.BlockSpec((1, _NB, _P), lambda b: (b, 0, 0)),
        out_shape=jax.ShapeDtypeStruct((_B, _NB, _P), jnp.float32),
    )(hist.reshape(_B, _NB, _P), pn_w.reshape(1, _P), pn_b.reshape(1, _P),
      gn_w, gn_b)


def kernel(events, pn_w, pn_b, gn_w, gn_b):
    idx = _compute_idx(events)
    hist = _hist_sc()(idx)
    return _postprocess(hist, pn_w, pn_b, gn_w, gn_b)
